# baseline (speedup 1.0000x reference)
import functools

import jax
import jax.numpy as jnp
from jax import lax
from jax.experimental import pallas as pl
from jax.experimental.pallas import tpu as pltpu
from jax.experimental.pallas import tpu_sc as plsc

NC = 2
NS = 16
NW = NC * NS
LANES = 16
IDX_COLS = 128
CHUNK_ROWS = 10


def _indirect_gather(table_hbm, idx_ref, dst_ref, sem):
    return pltpu.async_copy(table_hbm.at[idx_ref], dst_ref, sem)


def _worker_id():
    return lax.axis_index("s") * NC + lax.axis_index("c")


def _fix_group(extra_v, rows_ref, e, pos, m, d):
    def col_body(c, carry):
        cv = jnp.zeros((LANES,), jnp.int32) + c
        vals = plsc.load_gather(extra_v, [e, cv], mask=m)
        plsc.store_scatter(rows_ref, [pos, cv], vals, mask=m)
        return carry

    lax.fori_loop(0, d, col_body, 0)


@functools.lru_cache(maxsize=None)
def _build(n_tok, vocab, d):
    per_w = n_tok // NW
    idx_rows = per_w // IDX_COLS
    n_chunks = idx_rows // CHUNK_ROWS
    chunk_tok = CHUNK_ROWS * IDX_COLS
    assert per_w % IDX_COLS == 0 and idx_rows % CHUNK_ROWS == 0
    assert d % LANES == 0

    mesh = plsc.VectorSubcoreMesh(
        core_axis_name="c", subcore_axis_name="s",
        num_cores=NC, num_subcores=NS)

    def body(ids_hbm, table_hbm, extra_hbm, out_hbm,
             idx_v, rows0, rows1, extra_v, g0, g1, o0, o1):
        rows = (rows0, rows1)
        gsem = (g0, g1)
        osem = (o0, o1)
        wid = _worker_id()
        base = wid * per_w

        pltpu.sync_copy(ids_hbm.at[wid], idx_v)
        pltpu.sync_copy(extra_hbm, extra_v)

        def fire(g):
            b = g % 2
            return [
                _indirect_gather(
                    table_hbm, idx_v.at[g * CHUNK_ROWS + r],
                    rows[b].at[pl.ds(r * IDX_COLS, IDX_COLS)],
                    gsem[b])
                for r in range(CHUNK_ROWS)
            ]

        def reserved_hits(v):
            q = lax.shift_right_logical(v, 2)
            return ((q == 0) | (q == 25)).astype(jnp.int32)

        def fixup(g):
            b = g % 2

            def det_body(r, acc):
                row = g * CHUNK_ROWS + r
                for k in range(IDX_COLS // LANES):
                    v = idx_v[row, pl.ds(k * LANES, LANES)]
                    acc = acc | reserved_hits(v)
                return acc

            hits = lax.fori_loop(0, CHUNK_ROWS, det_body,
                                 jnp.zeros((LANES,), jnp.int32))

            @pl.when(jnp.max(hits) > 0)
            def _():
                def fix_row(r, carry):
                    row = g * CHUNK_ROWS + r
                    for k in range(IDX_COLS // LANES):
                        v = idx_v[row, pl.ds(k * LANES, LANES)]
                        hv = reserved_hits(v)
                        m = hv != 0

                        @pl.when(jnp.max(hv) > 0)
                        def _():
                            e = jnp.where(v < 4, v, v - 96)
                            e = jnp.clip(e, 0, 7)
                            pos = (r * IDX_COLS + k * LANES
                                   + lax.broadcasted_iota(jnp.int32, (LANES,), 0))
                            _fix_group(extra_v, rows[b], e, pos, m, d)
                    return carry

                lax.fori_loop(0, CHUNK_ROWS, fix_row, 0)

        gds = {0: fire(0)}
        outd = [None] * n_chunks
        for g in range(n_chunks):
            if g + 1 < n_chunks:
                if g >= 1:
                    outd[g - 1].wait()
                gds[g + 1] = fire(g + 1)
            for dsc in gds[g]:
                dsc.wait()
            fixup(g)
            b = g % 2
            outd[g] = pltpu.async_copy(
                rows[b],
                out_hbm.at[pl.ds(base + g * chunk_tok, chunk_tok)],
                osem[b])
        outd[n_chunks - 2].wait()
        outd[n_chunks - 1].wait()

    return pl.kernel(
        body,
        out_type=jax.ShapeDtypeStruct((n_tok, d), jnp.float32),
        mesh=mesh,
        compiler_params=pltpu.CompilerParams(
            use_tc_tiling_on_sc=False, needs_layout_passes=False),
        scratch_types=[
            pltpu.VMEM((idx_rows, IDX_COLS), jnp.int32),
            pltpu.VMEM((chunk_tok, d), jnp.float32),
            pltpu.VMEM((chunk_tok, d), jnp.float32),
            pltpu.VMEM((8, d), jnp.float32),
            pltpu.SemaphoreType.DMA,
            pltpu.SemaphoreType.DMA,
            pltpu.SemaphoreType.DMA,
            pltpu.SemaphoreType.DMA,
        ],
    )


def kernel(input_ids, weight, extra_embeddings):
    bsz, seq = input_ids.shape
    vocab, d = weight.shape
    n_tok = bsz * seq
    ids3d = input_ids.reshape(NW, n_tok // NW // IDX_COLS, IDX_COLS)
    out = _build(n_tok, vocab, d)(ids3d, weight, extra_embeddings)
    return out.reshape(bsz, seq, d)


# speedup vs baseline: 1.0019x; 1.0019x over previous
import functools

import jax
import jax.numpy as jnp
from jax import lax
from jax.experimental import pallas as pl
from jax.experimental.pallas import tpu as pltpu
from jax.experimental.pallas import tpu_sc as plsc

NC = 2
NS = 16
NW = NC * NS
LANES = 16
STREAM_LEN = 1280


def _indirect_gather(table_hbm, idx_ref, dst_ref, sem):
    return pltpu.async_copy(table_hbm.at[idx_ref], dst_ref, sem)


def _worker_id():
    return lax.axis_index("s") * NC + lax.axis_index("c")


def _fix_group(extra_v, rows_ref, e, pos, m, d):
    def col_body(c, carry):
        cv = jnp.zeros((LANES,), jnp.int32) + c
        vals = plsc.load_gather(extra_v, [e, cv], mask=m)
        plsc.store_scatter(rows_ref, [pos, cv], vals, mask=m)
        return carry

    lax.fori_loop(0, d, col_body, 0)


@functools.lru_cache(maxsize=None)
def _build(n_tok, vocab, d):
    per_w = n_tok // NW
    n_chunks = per_w // STREAM_LEN
    assert per_w % STREAM_LEN == 0 and n_chunks >= 2
    assert d % LANES == 0

    mesh = plsc.VectorSubcoreMesh(
        core_axis_name="c", subcore_axis_name="s",
        num_cores=NC, num_subcores=NS)

    def body(ids_hbm, table_hbm, extra_hbm, out_hbm,
             idx_v, rows0, rows1, extra_v, g0, g1, o0, o1):
        rows = (rows0, rows1)
        gsem = (g0, g1)
        osem = (o0, o1)
        wid = _worker_id()
        base = wid * per_w

        pltpu.sync_copy(ids_hbm.at[wid], idx_v)
        pltpu.sync_copy(extra_hbm, extra_v)

        def fire(g):
            b = g % 2
            return _indirect_gather(table_hbm, idx_v.at[g], rows[b], gsem[b])

        def reserved_hits(v):
            q = lax.shift_right_logical(v, 2)
            return ((q == 0) | (q == 25)).astype(jnp.int32)

        def detect(g):
            def det_body(j, acc):
                v = idx_v[g, pl.ds(j * LANES, LANES)]
                return acc | reserved_hits(v)

            return lax.fori_loop(0, STREAM_LEN // LANES, det_body,
                                 jnp.zeros((LANES,), jnp.int32))

        def fixup(g, hits):
            b = g % 2

            @pl.when(jnp.max(hits) > 0)
            def _():
                def fix_grp(j, carry):
                    v = idx_v[g, pl.ds(j * LANES, LANES)]
                    hv = reserved_hits(v)
                    m = hv != 0

                    @pl.when(jnp.max(hv) > 0)
                    def _():
                        e = jnp.where(v < 4, v, v - 96)
                        e = jnp.clip(e, 0, 7)
                        pos = (j * LANES
                               + lax.broadcasted_iota(jnp.int32, (LANES,), 0))
                        _fix_group(extra_v, rows[b], e, pos, m, d)
                    return carry

                lax.fori_loop(0, STREAM_LEN // LANES, fix_grp, 0)

        gds = {0: fire(0)}
        outd = [None] * n_chunks
        for g in range(n_chunks):
            if g + 1 < n_chunks:
                if g >= 1:
                    outd[g - 1].wait()
                gds[g + 1] = fire(g + 1)
            hits = detect(g)
            gds[g].wait()
            fixup(g, hits)
            b = g % 2
            outd[g] = pltpu.async_copy(
                rows[b],
                out_hbm.at[pl.ds(base + g * STREAM_LEN, STREAM_LEN)],
                osem[b])
        outd[n_chunks - 2].wait()
        outd[n_chunks - 1].wait()

    return pl.kernel(
        body,
        out_type=jax.ShapeDtypeStruct((n_tok, d), jnp.float32),
        mesh=mesh,
        compiler_params=pltpu.CompilerParams(
            use_tc_tiling_on_sc=False, needs_layout_passes=False),
        scratch_types=[
            pltpu.VMEM((per_w // STREAM_LEN, STREAM_LEN), jnp.int32),
            pltpu.VMEM((STREAM_LEN, d), jnp.float32),
            pltpu.VMEM((STREAM_LEN, d), jnp.float32),
            pltpu.VMEM((8, d), jnp.float32),
            pltpu.SemaphoreType.DMA,
            pltpu.SemaphoreType.DMA,
            pltpu.SemaphoreType.DMA,
            pltpu.SemaphoreType.DMA,
        ],
    )


def kernel(input_ids, weight, extra_embeddings):
    bsz, seq = input_ids.shape
    vocab, d = weight.shape
    n_tok = bsz * seq
    ids3d = input_ids.reshape(NW, n_tok // NW // STREAM_LEN, STREAM_LEN)
    out = _build(n_tok, vocab, d)(ids3d, weight, extra_embeddings)
    return out.reshape(bsz, seq, d)


# speedup vs baseline: 1.1599x; 1.1578x over previous
import functools

import jax
import jax.numpy as jnp
from jax import lax
from jax.experimental import pallas as pl
from jax.experimental.pallas import tpu as pltpu
from jax.experimental.pallas import tpu_sc as plsc

NC = 2
NS = 16
NW = NC * NS
LANES = 16


def _indirect_gather(table_hbm, idx_ref, dst_ref, sem):
    return pltpu.async_copy(table_hbm.at[idx_ref], dst_ref, sem)


def _worker_id():
    return lax.axis_index("s") * NC + lax.axis_index("c")


@functools.lru_cache(maxsize=None)
def _build(bsz, seq, vocab, d):
    bpw = bsz // NW
    assert bsz % NW == 0 and d == 32 and bpw % LANES == 0
    groups = bpw // LANES

    mesh = plsc.VectorSubcoreMesh(
        core_axis_name="c", subcore_axis_name="s",
        num_cores=NC, num_subcores=NS)

    def body(ids_hbm, table_hbm, extra_hbm, out_hbm,
             idx_v, idx4, rawA, rawB, outA, outB, extra_v,
             gsA, gsB, osA, osB):
        raw = (rawA, rawB)
        outb = (outA, outB)
        gsem = (gsA, gsB)
        osem = (osA, osB)
        wid = _worker_id()
        b0 = wid * bpw

        pltpu.sync_copy(ids_hbm.at[:, pl.ds(b0, bpw)], idx_v)
        pltpu.sync_copy(extra_hbm, extra_v)

        def mkidx(j, carry):
            s = j // groups
            k = j % groups
            v = idx_v[s, pl.ds(k * LANES, LANES)]
            idx4[s, pl.ds(k * LANES, LANES)] = lax.shift_right_logical(v, 2)
            return carry

        lax.fori_loop(0, seq * groups, mkidx, 0)

        def fire(s, b):
            return _indirect_gather(table_hbm, idx4.at[s], raw[b], gsem[b])

        def reserved_hits(v):
            q = lax.shift_right_logical(v, 2)
            return ((q == 0) | (q == 25)).astype(jnp.int32)

        def extract(s, b):
            def grp(t, carry):
                toks = t * LANES + lax.broadcasted_iota(jnp.int32, (LANES,), 0)
                v = idx_v[s, pl.ds(t * LANES, LANES)]
                colbase = lax.shift_left(v & 3, 5)
                for c in range(d):
                    vals = plsc.load_gather(raw[b], [toks, colbase + c])
                    outb[b][c, pl.ds(t * LANES, LANES)] = vals
                hv = reserved_hits(v)

                @pl.when(jnp.max(hv) > 0)
                def _():
                    m = hv != 0
                    e = jnp.where(v < 4, v, v - 96)
                    e = jnp.clip(e, 0, 7)
                    for c in range(d):
                        cv = jnp.zeros((LANES,), jnp.int32) + c
                        fv = plsc.load_gather(extra_v, [e, cv], mask=m)
                        plsc.store_scatter(outb[b], [cv, toks], fv, mask=m)
                return carry

            lax.fori_loop(0, groups, grp, 0)

        def out_dst(s):
            return out_hbm.at[s, :, pl.ds(b0, bpw)]

        def fire_out(s, b):
            return pltpu.async_copy(outb[b], out_dst(s), osem[b])

        def wait_out(s, b):
            pltpu.make_async_copy(outb[b], out_dst(s), osem[b]).wait()

        def chunk_pair(i, carry):
            s0 = i * 2
            s1 = s0 + 1

            @pl.when(i > 0)
            def _():
                wait_out(s1 - 2, 1)
            fire(s1, 1)
            pltpu.make_async_copy(
                table_hbm.at[idx4.at[s0]], raw[0], gsem[0]).wait()
            extract(s0, 0)
            fire_out(s0, 0)

            @pl.when(i < seq // 2 - 1)
            def _():
                wait_out(s0, 0)
                fire(s0 + 2, 0)
            pltpu.make_async_copy(
                table_hbm.at[idx4.at[s1]], raw[1], gsem[1]).wait()
            extract(s1, 1)
            fire_out(s1, 1)
            return carry

        fire(0, 0)
        lax.fori_loop(0, seq // 2, chunk_pair, 0)
        wait_out(seq - 2, 0)
        wait_out(seq - 1, 1)

    return pl.kernel(
        body,
        out_type=jax.ShapeDtypeStruct((seq, d, bsz), jnp.float32),
        mesh=mesh,
        compiler_params=pltpu.CompilerParams(
            use_tc_tiling_on_sc=True, needs_layout_passes=False),
        scratch_types=[
            pltpu.VMEM((seq, bpw), jnp.int32),
            pltpu.VMEM((seq, bpw), jnp.int32),
            pltpu.VMEM((bpw, 128), jnp.float32),
            pltpu.VMEM((bpw, 128), jnp.float32),
            pltpu.VMEM((d, bpw), jnp.float32),
            pltpu.VMEM((d, bpw), jnp.float32),
            pltpu.VMEM((8, d), jnp.float32),
            pltpu.SemaphoreType.DMA,
            pltpu.SemaphoreType.DMA,
            pltpu.SemaphoreType.DMA,
            pltpu.SemaphoreType.DMA,
        ],
    )


def kernel(input_ids, weight, extra_embeddings):
    bsz, seq = input_ids.shape
    vocab, d = weight.shape
    ids_t = input_ids.T
    w4 = weight.reshape(vocab // 4, 4 * d)
    p = _build(bsz, seq, vocab, d)(ids_t, w4, extra_embeddings)
    return p.transpose(2, 0, 1)
